# Initial kernel scaffold; baseline (speedup 1.0000x reference)
import functools

import jax
import jax.numpy as jnp
from jax import lax
from jax.experimental import pallas as pl
from jax.experimental.pallas import tpu as pltpu
from jax.experimental.pallas import tpu_sc as plsc

NUM_NODES = 1000000
E = 128
B = 16384
S = 6
L = 16
NC = 2
NS = 16
NW = NC * NS
B_PER_W = B // NW
CH = 64
NCHUNK = B_PER_W // CH
SIDX_ROWS = CH * S // E


def _logits_body(src_hbm, smp_hbm, node_hbm, ctx_hbm, out_hbm,
                 src_idx, smp_idx, src_rows, smp_rows, out_v, sem):
    wid = lax.axis_index("s") * NC + lax.axis_index("c")
    base = wid * B_PER_W

    def chunk_body(c, _):
        b0 = base + c * CH
        pltpu.sync_copy(src_hbm.at[pl.ds(b0, CH)], src_idx)
        pltpu.sync_copy(
            smp_hbm.at[pl.ds(b0 * S // E, SIDX_ROWS)], smp_idx)
        cp0 = pltpu.make_async_copy(node_hbm.at[src_idx], src_rows, sem)
        cp0.start()
        cps = []
        for j in range(SIDX_ROWS):
            cp = pltpu.make_async_copy(
                ctx_hbm.at[smp_idx.at[j]],
                smp_rows.at[pl.ds(j * E, E)], sem)
            cp.start()
            cps.append(cp)
        cp0.wait()
        for cp in cps:
            cp.wait()

        def b_body(bb, _):
            sv = [src_rows[bb, pl.ds(j * L, L)] for j in range(E // L)]
            for s in range(S):
                row = bb * S + s
                acc = sv[0] * smp_rows[row, pl.ds(0, L)]
                for j in range(1, E // L):
                    acc = acc + sv[j] * smp_rows[row, pl.ds(j * L, L)]
                out_v[row] = jnp.sum(acc)
            return 0

        lax.fori_loop(0, CH, b_body, 0, unroll=2)
        pltpu.sync_copy(out_v, out_hbm.at[pl.ds(b0 * S, CH * S)])
        return 0

    lax.fori_loop(0, NCHUNK, chunk_body, 0)


def _sc_logits(source_nodes, sample_rows_idx, node_embedding,
               context_embedding):
    mesh = plsc.VectorSubcoreMesh(core_axis_name="c", subcore_axis_name="s")
    return pl.kernel(
        _logits_body,
        out_type=jax.ShapeDtypeStruct((B * S,), jnp.float32),
        mesh=mesh,
        scratch_types=[
            pltpu.VMEM((CH,), jnp.int32),
            pltpu.VMEM((SIDX_ROWS, E), jnp.int32),
            pltpu.VMEM((CH, E), jnp.float32),
            pltpu.VMEM((CH * S, E), jnp.float32),
            pltpu.VMEM((CH * S,), jnp.float32),
            pltpu.SemaphoreType.DMA,
        ],
    )(source_nodes, sample_rows_idx, node_embedding, context_embedding)


def _loss_body(lg_ref, out_ref):
    x = lg_ref[...]
    r = lax.broadcasted_iota(jnp.int32, x.shape, 0)
    c = lax.broadcasted_iota(jnp.int32, x.shape, 1)
    is_pos = ((r * E + c) % S) == 0
    t = jnp.where(is_pos, x, -x)
    sp = jnp.maximum(-t, 0.0) + jnp.log1p(jnp.exp(-jnp.abs(t)))
    w = jnp.where(is_pos, 1.0 / B, 1.0 / (B * (S - 1)))
    out_ref[0, 0] = jnp.sum(sp * w)


def _tc_loss(logits2d):
    return pl.pallas_call(
        _loss_body,
        out_shape=jax.ShapeDtypeStruct((1, 1), jnp.float32),
        out_specs=pl.BlockSpec(memory_space=pltpu.SMEM),
    )(logits2d)


def kernel(source_nodes, sample_nodes, node_embedding, context_embedding):
    src = jnp.asarray(source_nodes, jnp.int32)
    smp = jnp.asarray(sample_nodes, jnp.int32).reshape(B * S // E, E)
    logits = _sc_logits(src, smp, node_embedding, context_embedding)
    loss = _tc_loss(logits.reshape(B * S // E, E))
    return loss[0, 0]


# speedup vs baseline: 14.4582x; 14.4582x over previous
import jax
import jax.numpy as jnp
from jax import lax
from jax.experimental import pallas as pl
from jax.experimental.pallas import tpu as pltpu
from jax.experimental.pallas import tpu_sc as plsc

NUM_NODES = 1000000
E = 128
B = 16384
S = 6
L = 16
NC = 2
NS = 16
NW = NC * NS
B_PER_W = B // NW
CH = 64
NCHUNK = B_PER_W // CH
SIDX_ROWS = CH * S // E


def _logits_body(src_hbm, smp_hbm, node_hbm, ctx_hbm, out_hbm,
                 src_idx, smp_idx, src_rows, smp_rows, out_v, sem):
    wid = lax.axis_index("s") * NC + lax.axis_index("c")
    base = wid * B_PER_W

    def chunk_body(c, _):
        b0 = base + c * CH
        pltpu.sync_copy(src_hbm.at[pl.ds(b0, CH)], src_idx)
        pltpu.sync_copy(smp_hbm.at[pl.ds(b0 * S, CH * S)], smp_idx)
        cp0 = pltpu.make_async_copy(node_hbm.at[src_idx], src_rows, sem)
        cp0.start()
        cps = []
        for j in range(SIDX_ROWS):
            cp = pltpu.make_async_copy(
                ctx_hbm.at[smp_idx.at[pl.ds(j * E, E)]],
                smp_rows.at[pl.ds(j * E, E)], sem)
            cp.start()
            cps.append(cp)
        cp0.wait()
        for cp in cps:
            cp.wait()

        lane = lax.iota(jnp.int32, L)
        last_lane = lane == (L - 1)

        def b_body(bb, _):
            sv = [src_rows[bb, pl.ds(j * L, L)] for j in range(E // L)]
            for s in range(S):
                row = bb * S + s
                acc = sv[0] * smp_rows[row, pl.ds(0, L)]
                for j in range(1, E // L):
                    acc = acc + sv[j] * smp_rows[row, pl.ds(j * L, L)]
                csum = plsc.cumsum(acc)
                plsc.store_scatter(
                    out_v, [jnp.full((L,), row, jnp.int32)], csum,
                    mask=last_lane)
            return 0

        lax.fori_loop(0, CH, b_body, 0, unroll=2)
        pltpu.sync_copy(out_v, out_hbm.at[pl.ds(b0 * S, CH * S)])
        return 0

    lax.fori_loop(0, NCHUNK, chunk_body, 0)


def _sc_logits(source_nodes, sample_rows_idx, node_embedding,
               context_embedding):
    mesh = plsc.VectorSubcoreMesh(
        core_axis_name="c", subcore_axis_name="s",
        num_cores=NC, num_subcores=NS)
    return pl.kernel(
        _logits_body,
        out_type=jax.ShapeDtypeStruct((B * S,), jnp.float32),
        mesh=mesh,
        scratch_types=[
            pltpu.VMEM((CH,), jnp.int32),
            pltpu.VMEM((CH * S,), jnp.int32),
            pltpu.VMEM((CH, E), jnp.float32),
            pltpu.VMEM((CH * S, E), jnp.float32),
            pltpu.VMEM((CH * S,), jnp.float32),
            pltpu.SemaphoreType.DMA,
        ],
        compiler_params=pltpu.CompilerParams(needs_layout_passes=False),
    )(source_nodes, sample_rows_idx, node_embedding, context_embedding)


def _loss_body(lg_ref, out_ref):
    x = lg_ref[...]
    r = lax.broadcasted_iota(jnp.int32, x.shape, 0)
    c = lax.broadcasted_iota(jnp.int32, x.shape, 1)
    is_pos = ((r * E + c) % S) == 0
    t = jnp.where(is_pos, x, -x)
    sp = jnp.maximum(-t, 0.0) + jnp.log1p(jnp.exp(-jnp.abs(t)))
    w = jnp.where(is_pos, 1.0 / B, 1.0 / (B * (S - 1)))
    out_ref[0, 0] = jnp.sum(sp * w)


def _tc_loss(logits2d):
    return pl.pallas_call(
        _loss_body,
        out_shape=jax.ShapeDtypeStruct((1, 1), jnp.float32),
        out_specs=pl.BlockSpec(memory_space=pltpu.SMEM),
    )(logits2d)


def kernel(source_nodes, sample_nodes, node_embedding, context_embedding):
    src = jnp.asarray(source_nodes, jnp.int32)
    smp = jnp.asarray(sample_nodes, jnp.int32).reshape(B * S)
    logits = _sc_logits(src, smp, node_embedding, context_embedding)
    loss = _tc_loss(logits.reshape(B * S // E, E))
    return loss[0, 0]


# speedup vs baseline: 16.8393x; 1.1647x over previous
import jax
import jax.numpy as jnp
from jax import lax
from jax.experimental import pallas as pl
from jax.experimental.pallas import tpu as pltpu
from jax.experimental.pallas import tpu_sc as plsc

NUM_NODES = 1000000
E = 128
B = 16384
S = 6
L = 16
NC = 2
NS = 16
NW = NC * NS
B_PER_W = B // NW
CH = 64
NCHUNK = B_PER_W // CH
SIDX_ROWS = CH * S // E


def _logits_body(src_hbm, smp_hbm, node_hbm, ctx_hbm, out_hbm,
                 src_idx, smp_idx, src_rows, smp_rows, out_v, sems):
    wid = lax.axis_index("s") * NC + lax.axis_index("c")
    base = wid * B_PER_W
    lane = lax.iota(jnp.int32, L)
    last_lane = lane == (L - 1)

    def stage_and_gather(c, p):
        b0 = base + c * CH
        pltpu.sync_copy(src_hbm.at[pl.ds(b0, CH)], src_idx.at[p])
        pltpu.sync_copy(smp_hbm.at[pl.ds(b0 * S, CH * S)], smp_idx.at[p])
        cps = [pltpu.make_async_copy(
            node_hbm.at[src_idx.at[p]], src_rows.at[p], sems.at[p])]
        for j in range(SIDX_ROWS):
            cps.append(pltpu.make_async_copy(
                ctx_hbm.at[smp_idx.at[p, pl.ds(j * E, E)]],
                smp_rows.at[p, pl.ds(j * E, E)], sems.at[p]))
        for cp in cps:
            cp.start()
        return cps

    cps = {0: stage_and_gather(0, 0)}
    for c in range(NCHUNK):
        p = c % 2
        if c + 1 < NCHUNK:
            cps[c + 1] = stage_and_gather(c + 1, (c + 1) % 2)
        for cp in cps.pop(c):
            cp.wait()

        def b_body(bb, _):
            sv = [src_rows[p, bb, pl.ds(j * L, L)] for j in range(E // L)]
            for s in range(S):
                row = bb * S + s
                acc = sv[0] * smp_rows[p, row, pl.ds(0, L)]
                for j in range(1, E // L):
                    acc = acc + sv[j] * smp_rows[p, row, pl.ds(j * L, L)]
                csum = plsc.cumsum(acc)
                plsc.store_scatter(
                    out_v, [jnp.full((L,), row, jnp.int32)], csum,
                    mask=last_lane)
            return 0

        lax.fori_loop(0, CH, b_body, 0, unroll=2)
        pltpu.sync_copy(out_v, out_hbm.at[pl.ds((base + c * CH) * S, CH * S)])


def _sc_logits(source_nodes, sample_rows_idx, node_embedding,
               context_embedding):
    mesh = plsc.VectorSubcoreMesh(
        core_axis_name="c", subcore_axis_name="s",
        num_cores=NC, num_subcores=NS)
    return pl.kernel(
        _logits_body,
        out_type=jax.ShapeDtypeStruct((B * S,), jnp.float32),
        mesh=mesh,
        scratch_types=[
            pltpu.VMEM((2, CH), jnp.int32),
            pltpu.VMEM((2, CH * S), jnp.int32),
            pltpu.VMEM((2, CH, E), jnp.float32),
            pltpu.VMEM((2, CH * S, E), jnp.float32),
            pltpu.VMEM((CH * S,), jnp.float32),
            pltpu.SemaphoreType.DMA((2,)),
        ],
        compiler_params=pltpu.CompilerParams(needs_layout_passes=False),
    )(source_nodes, sample_rows_idx, node_embedding, context_embedding)


def _loss_body(lg_ref, out_ref):
    x = lg_ref[...]
    r = lax.broadcasted_iota(jnp.int32, x.shape, 0)
    c = lax.broadcasted_iota(jnp.int32, x.shape, 1)
    is_pos = ((r * E + c) % S) == 0
    t = jnp.where(is_pos, x, -x)
    sp = jnp.maximum(-t, 0.0) + jnp.log1p(jnp.exp(-jnp.abs(t)))
    w = jnp.where(is_pos, 1.0 / B, 1.0 / (B * (S - 1)))
    out_ref[0, 0] = jnp.sum(sp * w)


def _tc_loss(logits2d):
    return pl.pallas_call(
        _loss_body,
        out_shape=jax.ShapeDtypeStruct((1, 1), jnp.float32),
        out_specs=pl.BlockSpec(memory_space=pltpu.SMEM),
    )(logits2d)


def kernel(source_nodes, sample_nodes, node_embedding, context_embedding):
    src = jnp.asarray(source_nodes, jnp.int32)
    smp = jnp.asarray(sample_nodes, jnp.int32).reshape(B * S)
    logits = _sc_logits(src, smp, node_embedding, context_embedding)
    loss = _tc_loss(logits.reshape(B * S // E, E))
    return loss[0, 0]
